# baseline (speedup 1.0000x reference)
import functools

import jax
import jax.numpy as jnp
from jax import lax
from jax.experimental import pallas as pl
from jax.experimental.pallas import tpu as pltpu
from jax.experimental.pallas import tpu_sc as plsc

N = 10000
E = 320000
D = 128
DE = 16
H = 256
EPS = 1e-7
BN_EPS = 1e-5

DH = D // 2
C = 64
NTILES = 16
CPS = 64
NSUP = 5
EPT = NSUP * CPS * C
E_PAD = NTILES * EPT
NP = 10112
RPT = NP // NTILES
BE = 512
NEB = E_PAD // BE
BN = 1000


def _edge_mm_body(ea_ref, we_ref, o_ref):
    o_ref[...] = jnp.dot(ea_ref[...], we_ref[0],
                         preferred_element_type=jnp.float32)


def _edge_mm(ea_pad, We):
    we2 = We.reshape(DE, 2, DH).transpose(1, 0, 2)
    return pl.pallas_call(
        _edge_mm_body,
        grid=(2, NEB),
        in_specs=[
            pl.BlockSpec((BE, DE), lambda c, i: (i, 0)),
            pl.BlockSpec((1, DE, DH), lambda c, i: (c, 0, 0)),
        ],
        out_specs=pl.BlockSpec((BE, DH), lambda c, i: (c * NEB + i, 0)),
        out_shape=jax.ShapeDtypeStruct((2 * E_PAD, DH), jnp.float32),
    )(ea_pad, we2)


_sc_mesh = plsc.VectorSubcoreMesh(core_axis_name="c", subcore_axis_name="s",
                                  num_cores=2, num_subcores=16)


@functools.partial(
    pl.kernel,
    out_type=jax.ShapeDtypeStruct((2 * NP, D), jnp.float32),
    mesh=_sc_mesh,
    compiler_params=pltpu.CompilerParams(use_tc_tiling_on_sc=False),
    scratch_types=[
        pltpu.VMEM((CPS, C), jnp.int32),
        pltpu.VMEM((4, C), jnp.int32),
        pltpu.VMEM((2, C, DH), jnp.float32),
        pltpu.VMEM((2, C, DH), jnp.float32),
        pltpu.VMEM((2, C, D), jnp.float32),
        pltpu.VMEM_SHARED((NP, D), jnp.float32),
        pltpu.SemaphoreType.DMA,
        pltpu.SemaphoreType.DMA,
        pltpu.SemaphoreType.DMA,
        pltpu.SemaphoreType.DMA,
        pltpu.SemaphoreType.DMA,
        pltpu.SemaphoreType.DMA,
        pltpu.SemaphoreType.DMA,
        pltpu.SemaphoreType.DMA,
    ],
)
def _sc_agg(x3, e3, src2_h, dst_h, out2,
            srcsup, dstv, gbuf, ebuf, obuf, acc,
            sg0, sg1, se0, se1, sd0, sd1, so0, so1):
    c = lax.axis_index("c")
    s = lax.axis_index("s")
    zero = jnp.zeros((16,), jnp.float32)
    nsplat = jnp.full((16,), N, jnp.int32)
    sg = (sg0, sg1)
    se = (se0, se1)
    sd = (sd0, sd1)
    so = (so0, so1)

    def zrow(i, carry):
        for j in range(D // 16):
            obuf[0, i, pl.ds(j * 16, 16)] = zero
            obuf[1, i, pl.ds(j * 16, 16)] = zero
        return carry

    lax.fori_loop(0, C, zrow, 0)
    for k in range(9):
        pltpu.sync_copy(obuf.at[0], acc.at[pl.ds(s * RPT + k * C, C)])
    pltpu.sync_copy(obuf.at[0, pl.ds(0, RPT - 576)],
                    acc.at[pl.ds(s * RPT + 576, RPT - 576)])
    for j in range(C // 16):
        dstv[0, pl.ds(j * 16, 16)] = nsplat
        dstv[1, pl.ds(j * 16, 16)] = nsplat
        dstv[2, pl.ds(j * 16, 16)] = nsplat
        dstv[3, pl.ds(j * 16, 16)] = nsplat
    plsc.subcore_barrier()
    pltpu.async_copy(obuf.at[0], acc.at[dstv.at[0]], so[0], add=True)
    pltpu.async_copy(obuf.at[1], acc.at[dstv.at[1]], so[1], add=True)

    ebase0 = s * EPT
    rbase0 = s * (NSUP * CPS)
    cN = c * N

    def adjust_src(i, carry):
        for j in range(C // 16):
            sl = pl.ds(j * 16, 16)
            srcsup[i, sl] = srcsup[i, sl] + cN
        return carry

    def issue(k, d, b):
        geb = ebase0_t + k * C
        pltpu.async_copy(dst_h.at[pl.ds(geb, C)], dstv.at[d], sd[b])
        pltpu.async_copy(x3.at[srcsup.at[k]], gbuf.at[b], sg[b])
        pltpu.async_copy(e3.at[pl.ds(cE + geb, C)], ebuf.at[b], se[b])

    def waitfor(k, d, b):
        geb = ebase0_t + k * C
        pltpu.make_async_copy(dst_h.at[pl.ds(geb, C)], dstv.at[d], sd[b]).wait()
        pltpu.make_async_copy(x3.at[srcsup.at[k]], gbuf.at[b], sg[b]).wait()
        pltpu.make_async_copy(e3.at[pl.ds(cE + geb, C)], ebuf.at[b], se[b]).wait()

    def do_chunk(k, d, b):
        pltpu.make_async_copy(obuf.at[b], acc.at[dstv.at[d]], so[b]).wait()

        @plsc.parallel_loop(0, C, step=1, unroll=4)
        def rowfn(r):
            for j in range(DH // 16):
                sl = pl.ds(j * 16, 16)
                m = jnp.maximum(gbuf[b, r, sl] + ebuf[b, r, sl], 0.0) + EPS
                t = jnp.exp(m)
                obuf[b, r, pl.ds(j * 16, 16)] = t
                obuf[b, r, pl.ds(DH + j * 16, 16)] = m * t

        pltpu.async_copy(obuf.at[b], acc.at[dstv.at[d]], so[b], add=True)

    cE = c * E_PAD
    for t in range(NSUP):
        ebase0_t = ebase0 + t * CPS * C
        pltpu.sync_copy(src2_h.at[pl.ds(rbase0 + t * CPS, CPS)], srcsup)
        lax.fori_loop(0, CPS, adjust_src, 0)
        issue(0, 0, 0)

        def quad(q, carry):
            k0 = 4 * q
            for b in range(4):
                k = k0 + b
                knext = k + 1 if b < 3 else jnp.minimum(k + 1, CPS - 1)
                issue(knext, (b + 1) % 4, (b + 1) % 2)
                waitfor(k, b, b % 2)
                do_chunk(k, b, b % 2)
            return carry

        lax.fori_loop(0, CPS // 4, quad, 0)
        waitfor(CPS - 1, 0, 0)

    pltpu.make_async_copy(obuf.at[0], acc.at[dstv.at[0]], so[0]).wait()
    pltpu.make_async_copy(obuf.at[1], acc.at[dstv.at[1]], so[1]).wait()
    plsc.subcore_barrier()

    for k in range(9):
        off = s * RPT + k * C
        pltpu.sync_copy(acc.at[pl.ds(off, C)], obuf.at[0])
        pltpu.sync_copy(obuf.at[0], out2.at[pl.ds(c * NP + off, C)])
    off = s * RPT + 576
    vb = obuf.at[0, pl.ds(0, RPT - 576)]
    pltpu.sync_copy(acc.at[pl.ds(off, RPT - 576)], vb)
    pltpu.sync_copy(vb, out2.at[pl.ds(c * NP + off, RPT - 576)])


def _node_mlp_body(relu_out, o2a_ref, o2b_ref, x_ref, wa_ref, s1_ref, b1_ref,
                   wb_ref, o_ref):
    a = o2a_ref[0]
    b = o2b_ref[0]
    den = jnp.concatenate([a[:, :DH], b[:, :DH]], axis=1)
    num = jnp.concatenate([a[:, DH:], b[:, DH:]], axis=1)
    agg = num / jnp.where(den == 0.0, 1.0, den)
    o = agg + x_ref[...]
    h = jnp.dot(o, wa_ref[...], preferred_element_type=jnp.float32)
    h = jnp.maximum(h * s1_ref[...] + b1_ref[...], 0.0)
    y = jnp.dot(h, wb_ref[...], preferred_element_type=jnp.float32)
    if relu_out:
        y = jnp.maximum(y, 0.0)
    o_ref[...] = y


def _node_mlp(out2, x, Wa, s1, b1, Wb, relu_out):
    out2v = out2.reshape(2, NP, D)
    return pl.pallas_call(
        functools.partial(_node_mlp_body, relu_out),
        grid=(N // BN,),
        in_specs=[
            pl.BlockSpec((1, BN, D), lambda i: (0, i, 0)),
            pl.BlockSpec((1, BN, D), lambda i: (1, i, 0)),
            pl.BlockSpec((BN, D), lambda i: (i, 0)),
            pl.BlockSpec((D, H), lambda i: (0, 0)),
            pl.BlockSpec((1, H), lambda i: (0, 0)),
            pl.BlockSpec((1, H), lambda i: (0, 0)),
            pl.BlockSpec((H, D), lambda i: (0, 0)),
        ],
        out_specs=pl.BlockSpec((BN, D), lambda i: (i, 0)),
        out_shape=jax.ShapeDtypeStruct((N, D), jnp.float32),
    )(out2v, out2v, x, Wa, s1, b1, Wb)


def _layer(xin, src2, dst, ea_pad, We, Wa, bnw, bnb, Wb, relu_out):
    e3 = _edge_mm(ea_pad, We)
    x3 = jnp.concatenate([xin[:, :DH], xin[:, DH:]], axis=0)
    out2 = _sc_agg(x3, e3, src2, dst)
    s1 = (bnw / jnp.sqrt(1.0 + BN_EPS)).reshape(1, H)
    b1 = bnb.reshape(1, H)
    return _node_mlp(out2, xin, Wa, s1, b1, Wb, relu_out)


def kernel(x, edge_index, edge_attr, We1, W1a, bn1w, bn1b, W1b,
           We2, W2a, bn2w, bn2b, W2b):
    pad = E_PAD - E
    src2 = jnp.concatenate(
        [edge_index[0], jnp.zeros((pad,), jnp.int32)]).reshape(E_PAD // C, C)
    dst = jnp.concatenate([edge_index[1], jnp.full((pad,), N, jnp.int32)])
    ea_pad = jnp.concatenate(
        [edge_attr, jnp.zeros((pad, DE), jnp.float32)], axis=0)
    h = _layer(x, src2, dst, ea_pad, We1, W1a, bn1w, bn1b, W1b, True)
    return _layer(h, src2, dst, ea_pad, We2, W2a, bn2w, bn2b, W2b, False)
